# Initial kernel scaffold; baseline (speedup 1.0000x reference)
import jax
import jax.numpy as jnp
from jax.experimental import pallas as pl


def kernel(X, edge_index, W1, b1, W2, b2, Wc, bc):
    raise NotImplementedError("write your pallas kernel here")


# speedup vs baseline: 12.9857x; 12.9857x over previous
import functools

import jax
import jax.numpy as jnp
from jax import lax
from jax.experimental import pallas as pl
from jax.experimental.pallas import tpu as pltpu
from jax.experimental.pallas import tpu_sc as plsc

NC = 2
NS = 16
NP = 10240
RPT = NP // NS
K = 80
BM = 640
DW = 128


def _sc_mesh():
    return plsc.VectorSubcoreMesh(
        core_axis_name="c", subcore_axis_name="s", num_cores=NC,
        num_subcores=NS)


def _zero_vmem_1d(ref, n):
    def body(i, _):
        ref[pl.ds(i * 16, 16)] = jnp.zeros((16,), jnp.float32)
        return 0
    lax.fori_loop(0, n // 16, body, 0)


def _zero_vmem_2d(ref, rows, cols):
    def body(i, _):
        r = i // (cols // 16)
        j = lax.rem(i, cols // 16)
        ref[r, pl.ds(j * 16, 16)] = jnp.zeros((16,), jnp.float32)
        return 0
    lax.fori_loop(0, rows * (cols // 16), body, 0)


def _make_deg_kernel(E):
    EPC = E // NC
    EPT = EPC // NS
    NB = EPT // K

    @functools.partial(
        pl.kernel,
        out_type=jax.ShapeDtypeStruct((NC, NP), jnp.float32),
        mesh=_sc_mesh(),
        scratch_types=[
            pltpu.VMEM_SHARED((NP,), jnp.float32),
            pltpu.VMEM((RPT,), jnp.float32),
            pltpu.VMEM((K,), jnp.int32),
            pltpu.VMEM((K,), jnp.float32),
        ],
    )
    def deg_k(dst_hbm, out_hbm, acc_s, zbuf, idx_v, ones_v):
        c = lax.axis_index("c")
        s = lax.axis_index("s")
        _zero_vmem_1d(zbuf, RPT)

        def ones(i, _):
            ones_v[pl.ds(i * 16, 16)] = jnp.ones((16,), jnp.float32)
            return 0
        lax.fori_loop(0, K // 16, ones, 0)

        pltpu.sync_copy(zbuf, acc_s.at[pl.ds(s * RPT, RPT)])
        plsc.subcore_barrier()

        base = c * EPC + s * EPT

        def body(i, _):
            pltpu.sync_copy(dst_hbm.at[pl.ds(base + i * K, K)], idx_v)
            pltpu.sync_copy(ones_v, acc_s.at[idx_v], add=True)
            return 0
        lax.fori_loop(0, NB, body, 0)

        plsc.subcore_barrier()
        pltpu.sync_copy(acc_s.at[pl.ds(s * RPT, RPT)],
                        out_hbm.at[c, pl.ds(s * RPT, RPT)])

    return deg_k


def _make_scatter_kernel(E):
    EPT = E // (NC * NS)
    NB = EPT // K

    @functools.partial(
        pl.kernel,
        out_type=jax.ShapeDtypeStruct((NC, NP, DW), jnp.float32),
        mesh=_sc_mesh(),
        scratch_types=[
            pltpu.VMEM_SHARED((NP, DW), jnp.float32),
            pltpu.VMEM((K,), jnp.int32),
            pltpu.VMEM((K,), jnp.int32),
            pltpu.VMEM((K, DW), jnp.float32),
            pltpu.VMEM((K, DW), jnp.float32),
            pltpu.SemaphoreType.DMA,
        ],
    )
    def scat_k(h_hbm, src_hbm, dst_hbm, out_hbm,
               acc_s, sidx, didx, msg, zbuf, sem):
        c = lax.axis_index("c")
        s = lax.axis_index("s")

        _zero_vmem_2d(zbuf, K, DW)
        for j in range(RPT // K):
            pltpu.sync_copy(zbuf, acc_s.at[pl.ds(s * RPT + j * K, K)])
        plsc.subcore_barrier()

        base = c * (E // NC) + s * EPT

        def body(i, _):
            pltpu.sync_copy(src_hbm.at[pl.ds(base + i * K, K)], sidx)
            pltpu.sync_copy(dst_hbm.at[pl.ds(base + i * K, K)], didx)
            pltpu.async_copy(h_hbm.at[sidx], msg, sem).wait()
            pltpu.sync_copy(msg, acc_s.at[didx], add=True)
            return 0
        lax.fori_loop(0, NB, body, 0)

        plsc.subcore_barrier()
        pltpu.sync_copy(acc_s.at[pl.ds(s * RPT, RPT)],
                        out_hbm.at[c, pl.ds(s * RPT, RPT)])

    return scat_k


def _mm1(XP, W1, counts):
    def body(x_ref, w_ref, cnt_ref, h_ref, dis_ref):
        deg = cnt_ref[0, :] + cnt_ref[1, :] + 1.0
        dis = lax.rsqrt(deg)
        h = jnp.dot(x_ref[...], w_ref[...],
                    preferred_element_type=jnp.float32)
        h_ref[...] = h * dis[:, None]
        dis_ref[...] = dis[:, None]

    return pl.pallas_call(
        body,
        grid=(NP // BM,),
        in_specs=[
            pl.BlockSpec((BM, 128), lambda i: (i, 0)),
            pl.BlockSpec((128, 128), lambda i: (0, 0)),
            pl.BlockSpec((NC, BM), lambda i: (0, i)),
        ],
        out_specs=[
            pl.BlockSpec((BM, 128), lambda i: (i, 0)),
            pl.BlockSpec((BM, 1), lambda i: (i, 0)),
        ],
        out_shape=[
            jax.ShapeDtypeStruct((NP, 128), jnp.float32),
            jax.ShapeDtypeStruct((NP, 1), jnp.float32),
        ],
    )(XP, W1, counts)


def _mm2(agg1, h1p, dis, b1, W2):
    def body(a_ref, h_ref, dis_ref, b_ref, w_ref, o_ref):
        dis = dis_ref[...]
        x2 = jnp.maximum(
            dis * (a_ref[0] + a_ref[1] + h_ref[...]) + b_ref[0, :], 0.0)
        h2 = jnp.dot(x2, w_ref[...], preferred_element_type=jnp.float32)
        h2 = h2 * dis
        o_ref[...] = jnp.concatenate(
            [h2, jnp.zeros_like(h2)], axis=1)

    return pl.pallas_call(
        body,
        grid=(NP // BM,),
        in_specs=[
            pl.BlockSpec((NC, BM, 128), lambda i: (0, i, 0)),
            pl.BlockSpec((BM, 128), lambda i: (i, 0)),
            pl.BlockSpec((BM, 1), lambda i: (i, 0)),
            pl.BlockSpec((1, 128), lambda i: (0, 0)),
            pl.BlockSpec((128, 64), lambda i: (0, 0)),
        ],
        out_specs=pl.BlockSpec((BM, 128), lambda i: (i, 0)),
        out_shape=jax.ShapeDtypeStruct((NP, 128), jnp.float32),
    )(agg1, h1p, dis, b1, W2)


def _mm3(agg2, h2p, dis, b2, Wc, bc):
    def body(a_ref, h_ref, dis_ref, b_ref, w_ref, bc_ref, o_ref):
        dis = dis_ref[...]
        t = (a_ref[0] + a_ref[1] + h_ref[...])[:, :64]
        x3 = jnp.maximum(dis * t + b_ref[0, :], 0.0)
        o_ref[...] = jnp.dot(
            x3, w_ref[...], preferred_element_type=jnp.float32) + bc_ref[0, :]

    return pl.pallas_call(
        body,
        grid=(NP // BM,),
        in_specs=[
            pl.BlockSpec((NC, BM, 128), lambda i: (0, i, 0)),
            pl.BlockSpec((BM, 128), lambda i: (i, 0)),
            pl.BlockSpec((BM, 1), lambda i: (i, 0)),
            pl.BlockSpec((1, 64), lambda i: (0, 0)),
            pl.BlockSpec((64, 16), lambda i: (0, 0)),
            pl.BlockSpec((1, 16), lambda i: (0, 0)),
        ],
        out_specs=pl.BlockSpec((BM, 16), lambda i: (i, 0)),
        out_shape=jax.ShapeDtypeStruct((NP, 16), jnp.float32),
    )(agg2, h2p, dis, b2, Wc, bc)


def kernel(X, edge_index, W1, b1, W2, b2, Wc, bc):
    N = X.shape[0]
    E = edge_index.shape[1]
    src = edge_index[0].astype(jnp.int32)
    dst = edge_index[1].astype(jnp.int32)

    XP = jnp.pad(X, ((0, NP - N), (0, 0)))

    counts = _make_deg_kernel(E)(dst)
    h1p, dis = _mm1(XP, W1, counts)
    scat = _make_scatter_kernel(E)
    agg1 = scat(h1p, src, dst)
    h2p = _mm2(agg1, h1p, dis, b1.reshape(1, -1), W2)
    agg2 = scat(h2p, src, dst)
    logits = _mm3(agg2, h2p, dis, b2.reshape(1, -1), Wc, bc.reshape(1, -1))
    return logits[:N]


# speedup vs baseline: 26.4461x; 2.0366x over previous
import functools

import jax
import jax.numpy as jnp
from jax import lax
from jax.experimental import pallas as pl
from jax.experimental.pallas import tpu as pltpu
from jax.experimental.pallas import tpu_sc as plsc

NC = 2
NS = 16
NW = NC * NS
NP = 10240
RPT = NP // NS
K = 128
BM = 640
DW = 128


def _sc_mesh():
    return plsc.VectorSubcoreMesh(
        core_axis_name="c", subcore_axis_name="s", num_cores=NC,
        num_subcores=NS)


def _zero_vmem_1d(ref, n):
    def body(i, _):
        ref[pl.ds(i * 16, 16)] = jnp.zeros((16,), jnp.float32)
        return 0
    lax.fori_loop(0, n // 16, body, 0)


def _zero_vmem_2d(ref, rows, cols):
    def body(i, _):
        r = i // (cols // 16)
        j = lax.rem(i, cols // 16)
        ref[r, pl.ds(j * 16, 16)] = jnp.zeros((16,), jnp.float32)
        return 0
    lax.fori_loop(0, rows * (cols // 16), body, 0)


def _wait(src, dst, sem, add=False):
    del add
    pltpu.make_async_copy(src, dst, sem).wait()


def _make_deg_kernel(NB):
    @functools.partial(
        pl.kernel,
        out_type=jax.ShapeDtypeStruct((NC, NP), jnp.float32),
        mesh=_sc_mesh(),
        scratch_types=[
            pltpu.VMEM_SHARED((NP,), jnp.float32),
            pltpu.VMEM((NB, K), jnp.int32),
            pltpu.VMEM((K,), jnp.float32),
            pltpu.VMEM((RPT,), jnp.float32),
            pltpu.SemaphoreType.DMA,
            pltpu.SemaphoreType.DMA,
        ],
    )
    def deg_k(dst2_hbm, out_hbm, acc_s, idx2, ones_v, zbuf, lsem, sem):
        c = lax.axis_index("c")
        s = lax.axis_index("s")
        wid = c * NS + s

        ld = pltpu.async_copy(dst2_hbm.at[pl.ds(wid * NB, NB)], idx2, lsem)
        _zero_vmem_1d(zbuf, RPT)

        def ones(i, _):
            ones_v[pl.ds(i * 16, 16)] = jnp.ones((16,), jnp.float32)
            return 0
        lax.fori_loop(0, K // 16, ones, 0)

        pltpu.sync_copy(zbuf, acc_s.at[pl.ds(s * RPT, RPT)])
        ld.wait()
        plsc.subcore_barrier()

        def body(i, _):
            pltpu.async_copy(ones_v, acc_s.at[idx2.at[i]], sem, add=True)
            return 0
        lax.fori_loop(0, NB, body, 0)

        def drain(i, _):
            _wait(ones_v, acc_s.at[idx2.at[i]], sem, add=True)
            return 0
        lax.fori_loop(0, NB, drain, 0)

        plsc.subcore_barrier()
        pltpu.sync_copy(acc_s.at[pl.ds(s * RPT, RPT)],
                        out_hbm.at[c, pl.ds(s * RPT, RPT)])

    return deg_k


def _make_scatter_kernel(NB):
    CH = 5
    G = NB // CH
    NJ = G // 2

    @functools.partial(
        pl.kernel,
        out_type=jax.ShapeDtypeStruct((NC, NP, DW), jnp.float32),
        mesh=_sc_mesh(),
        scratch_types=[
            pltpu.VMEM_SHARED((NP, DW), jnp.float32),
            pltpu.VMEM((G, K), jnp.int32),
            pltpu.VMEM((G, K), jnp.int32),
            pltpu.VMEM((K, DW), jnp.float32),
            pltpu.VMEM((K, DW), jnp.float32),
            pltpu.SemaphoreType.DMA,
            pltpu.SemaphoreType.DMA,
            pltpu.SemaphoreType.DMA,
            pltpu.SemaphoreType.DMA,
            pltpu.SemaphoreType.DMA,
        ],
    )
    def scat_k(h_hbm, src2_hbm, dst2_hbm, out_hbm,
               acc_s, sidx2, didx2, msg0, msg1, lsem, g0, g1, s0, s1):
        c = lax.axis_index("c")
        s = lax.axis_index("s")
        wid = c * NS + s

        _zero_vmem_2d(msg0, K, DW)
        for j in range(RPT // K):
            pltpu.sync_copy(msg0, acc_s.at[pl.ds(s * RPT + j * K, K)])
        plsc.subcore_barrier()

        for ch in range(CH):
            row0 = wid * NB + ch * G
            lc1 = pltpu.async_copy(src2_hbm.at[pl.ds(row0, G)], sidx2, lsem)
            lc2 = pltpu.async_copy(dst2_hbm.at[pl.ds(row0, G)], didx2, lsem)
            lc1.wait()
            lc2.wait()

            pltpu.async_copy(h_hbm.at[sidx2.at[0]], msg0, g0)

            def pair(j, _):
                i = 2 * j
                _wait(h_hbm.at[sidx2.at[i]], msg0, g0)

                @pl.when(j > 0)
                def _():
                    _wait(msg1, acc_s.at[didx2.at[i - 1]], s1)

                pltpu.async_copy(h_hbm.at[sidx2.at[i + 1]], msg1, g1)
                pltpu.async_copy(msg0, acc_s.at[didx2.at[i]], s0, add=True)
                _wait(h_hbm.at[sidx2.at[i + 1]], msg1, g1)
                _wait(msg0, acc_s.at[didx2.at[i]], s0)

                @pl.when(j < NJ - 1)
                def _():
                    pltpu.async_copy(h_hbm.at[sidx2.at[i + 2]], msg0, g0)

                pltpu.async_copy(msg1, acc_s.at[didx2.at[i + 1]], s1,
                                 add=True)
                return 0
            lax.fori_loop(0, NJ, pair, 0)
            _wait(msg1, acc_s.at[didx2.at[G - 1]], s1)

        plsc.subcore_barrier()
        pltpu.sync_copy(acc_s.at[pl.ds(s * RPT, RPT)],
                        out_hbm.at[c, pl.ds(s * RPT, RPT)])

    return scat_k


def _mm_a(XP, W1):
    def body(x_ref, w_ref, h_ref):
        h_ref[...] = jnp.dot(x_ref[...], w_ref[...],
                             preferred_element_type=jnp.float32)

    return pl.pallas_call(
        body,
        grid=(NP // BM,),
        in_specs=[
            pl.BlockSpec((BM, 128), lambda i: (i, 0)),
            pl.BlockSpec((128, 128), lambda i: (0, 0)),
        ],
        out_specs=pl.BlockSpec((BM, 128), lambda i: (i, 0)),
        out_shape=jax.ShapeDtypeStruct((NP, 128), jnp.float32),
    )(XP, W1)


def _mm_scale(h1, counts):
    def body(h_ref, cnt_ref, hp_ref, dis_ref):
        deg = cnt_ref[0, :] + cnt_ref[1, :] + 1.0
        dis = lax.rsqrt(deg)
        hp_ref[...] = h_ref[...] * dis[:, None]
        dis_ref[...] = dis[:, None]

    return pl.pallas_call(
        body,
        grid=(NP // BM,),
        in_specs=[
            pl.BlockSpec((BM, 128), lambda i: (i, 0)),
            pl.BlockSpec((NC, BM), lambda i: (0, i)),
        ],
        out_specs=[
            pl.BlockSpec((BM, 128), lambda i: (i, 0)),
            pl.BlockSpec((BM, 1), lambda i: (i, 0)),
        ],
        out_shape=[
            jax.ShapeDtypeStruct((NP, 128), jnp.float32),
            jax.ShapeDtypeStruct((NP, 1), jnp.float32),
        ],
    )(h1, counts)


def _mm2(agg1, h1p, dis, b1, W2):
    def body(a_ref, h_ref, dis_ref, b_ref, w_ref, o_ref):
        dis = dis_ref[...]
        x2 = jnp.maximum(
            dis * (a_ref[0] + a_ref[1] + h_ref[...]) + b_ref[0, :], 0.0)
        h2 = jnp.dot(x2, w_ref[...], preferred_element_type=jnp.float32)
        h2 = h2 * dis
        o_ref[...] = jnp.concatenate([h2, jnp.zeros_like(h2)], axis=1)

    return pl.pallas_call(
        body,
        grid=(NP // BM,),
        in_specs=[
            pl.BlockSpec((NC, BM, 128), lambda i: (0, i, 0)),
            pl.BlockSpec((BM, 128), lambda i: (i, 0)),
            pl.BlockSpec((BM, 1), lambda i: (i, 0)),
            pl.BlockSpec((1, 128), lambda i: (0, 0)),
            pl.BlockSpec((128, 64), lambda i: (0, 0)),
        ],
        out_specs=pl.BlockSpec((BM, 128), lambda i: (i, 0)),
        out_shape=jax.ShapeDtypeStruct((NP, 128), jnp.float32),
    )(agg1, h1p, dis, b1, W2)


def _mm3(agg2, h2p, dis, b2, Wc, bc):
    def body(a_ref, h_ref, dis_ref, b_ref, w_ref, bc_ref, o_ref):
        dis = dis_ref[...]
        t = (a_ref[0] + a_ref[1] + h_ref[...])[:, :64]
        x3 = jnp.maximum(dis * t + b_ref[0, :], 0.0)
        o_ref[...] = jnp.dot(
            x3, w_ref[...], preferred_element_type=jnp.float32) + bc_ref[0, :]

    return pl.pallas_call(
        body,
        grid=(NP // BM,),
        in_specs=[
            pl.BlockSpec((NC, BM, 128), lambda i: (0, i, 0)),
            pl.BlockSpec((BM, 128), lambda i: (i, 0)),
            pl.BlockSpec((BM, 1), lambda i: (i, 0)),
            pl.BlockSpec((1, 64), lambda i: (0, 0)),
            pl.BlockSpec((64, 16), lambda i: (0, 0)),
            pl.BlockSpec((1, 16), lambda i: (0, 0)),
        ],
        out_specs=pl.BlockSpec((BM, 16), lambda i: (i, 0)),
        out_shape=jax.ShapeDtypeStruct((NP, 16), jnp.float32),
    )(agg2, h2p, dis, b2, Wc, bc)


def _pad_edges(idx, E):
    ept = E // NW
    eptp = pl.cdiv(ept, 2 * K) * 2 * K
    pad = jnp.broadcast_to(
        10000 + jnp.arange(eptp - ept, dtype=jnp.int32) % (NP - 10000),
        (NW, eptp - ept))
    full = jnp.concatenate([idx.reshape(NW, ept), pad], axis=1)
    return full.reshape(NW * (eptp // K), K), eptp // K


def kernel(X, edge_index, W1, b1, W2, b2, Wc, bc):
    N = X.shape[0]
    E = edge_index.shape[1]
    src = edge_index[0].astype(jnp.int32)
    dst = edge_index[1].astype(jnp.int32)
    src2, NB = _pad_edges(src, E)
    dst2, _ = _pad_edges(dst, E)

    XP = jnp.pad(X, ((0, NP - N), (0, 0)))

    counts = _make_deg_kernel(NB)(dst2)
    h1 = _mm_a(XP, W1)
    h1p, dis = _mm_scale(h1, counts)
    scat = _make_scatter_kernel(NB)
    agg1 = scat(h1p, src2, dst2)
    h2p = _mm2(agg1, h1p, dis, b1.reshape(1, -1), W2)
    agg2 = scat(h2p, src2, dst2)
    logits = _mm3(agg2, h2p, dis, b2.reshape(1, -1), Wc, bc.reshape(1, -1))
    return logits[:N]
